# baseline (speedup 1.0000x reference)
import functools

import jax
import jax.numpy as jnp
from jax import lax
from jax.experimental import pallas as pl
from jax.experimental.pallas import tpu as pltpu
from jax.experimental.pallas import tpu_sc as plsc

_NC = 2
_NS = 16
_NW = _NC * _NS
_B = 16384
_D = 64
_RPW = _B // _NW
_CHUNK = 128
_NCHUNK = _RPW // _CHUNK
_REG = 0.0001


def _sc_body(u_hbm, pi_hbm, ni_hbm, ue_hbm, ie_hbm, diff_hbm, sq_hbm,
             idx_u, idx_p, idx_n, ue_v, pe_v, ne_v, diff_v, sq_v, sem):
    wid = lax.axis_index("s") * _NC + lax.axis_index("c")

    pltpu.sync_copy(u_hbm.at[wid], idx_u)
    pltpu.sync_copy(pi_hbm.at[wid], idx_p)
    pltpu.sync_copy(ni_hbm.at[wid], idx_n)

    copies = []
    for c in range(_NCHUNK):
        dst = pl.ds(c * _CHUNK, _CHUNK)
        copies.append(pltpu.async_copy(ue_hbm.at[idx_u.at[c]], ue_v.at[dst], sem))
        copies.append(pltpu.async_copy(ie_hbm.at[idx_p.at[c]], pe_v.at[dst], sem))
        copies.append(pltpu.async_copy(ie_hbm.at[idx_n.at[c]], ne_v.at[dst], sem))
    for cp in copies:
        cp.wait()
    lanes = lax.iota(jnp.int32, 16)

    def group_body(g, acc_sq):
        rows = g * 16 + lanes

        def d_body(d, carry):
            acc_d, acc_sq = carry
            col = jnp.zeros((16,), jnp.int32) + d
            uev = plsc.load_gather(ue_v, [rows, col])
            pev = plsc.load_gather(pe_v, [rows, col])
            nev = plsc.load_gather(ne_v, [rows, col])
            acc_d = acc_d + uev * (pev - nev)
            acc_sq = acc_sq + uev * uev + pev * pev + nev * nev
            return acc_d, acc_sq

        acc_d, acc_sq = lax.fori_loop(
            0, _D, d_body, (jnp.zeros((16,), jnp.float32), acc_sq))
        diff_v[pl.ds(g * 16, 16)] = acc_d
        return acc_sq

    acc_sq = lax.fori_loop(0, _RPW // 16, group_body,
                           jnp.zeros((16,), jnp.float32))
    sq_v[...] = acc_sq
    pltpu.sync_copy(diff_v, diff_hbm.at[wid])
    pltpu.sync_copy(sq_v, sq_hbm.at[wid])


_sc_call = pl.kernel(
    _sc_body,
    out_type=(
        jax.ShapeDtypeStruct((_NW, _RPW), jnp.float32),
        jax.ShapeDtypeStruct((_NW, 16), jnp.float32),
    ),
    mesh=plsc.VectorSubcoreMesh(core_axis_name="c", subcore_axis_name="s"),
    compiler_params=pltpu.CompilerParams(needs_layout_passes=False, use_tc_tiling_on_sc=False),
    scratch_types=[
        pltpu.VMEM((_NCHUNK, _CHUNK), jnp.int32),
        pltpu.VMEM((_NCHUNK, _CHUNK), jnp.int32),
        pltpu.VMEM((_NCHUNK, _CHUNK), jnp.int32),
        pltpu.VMEM((_RPW, _D), jnp.float32),
        pltpu.VMEM((_RPW, _D), jnp.float32),
        pltpu.VMEM((_RPW, _D), jnp.float32),
        pltpu.VMEM((_RPW,), jnp.float32),
        pltpu.VMEM((16,), jnp.float32),
        pltpu.SemaphoreType.DMA,
    ],
)


def _tc_body(diff_ref, sq_ref, out_ref):
    x = diff_ref[...]
    s = 1.0 / (1.0 + jnp.exp(-x))
    l = jnp.log(s + 1e-10)
    bpr = -jnp.sum(l) / _B
    reg = _REG * jnp.sum(sq_ref[...]) / _B
    out_ref[0, 0] = bpr + reg


def kernel(u, pi, ni, user_emb, item_emb):
    u3 = u.reshape(_NW, _NCHUNK, _CHUNK)
    pi3 = pi.reshape(_NW, _NCHUNK, _CHUNK)
    ni3 = ni.reshape(_NW, _NCHUNK, _CHUNK)
    diff, sq = _sc_call(u3, pi3, ni3, user_emb, item_emb)
    out = pl.pallas_call(
        _tc_body,
        out_shape=jax.ShapeDtypeStruct((1, 1), jnp.float32),
        in_specs=[
            pl.BlockSpec(memory_space=pltpu.VMEM),
            pl.BlockSpec(memory_space=pltpu.VMEM),
        ],
        out_specs=pl.BlockSpec(memory_space=pltpu.SMEM),
    )(diff.reshape(128, 128), sq.reshape(4, 128))
    return out.reshape(())


# speedup vs baseline: 2.7974x; 2.7974x over previous
import jax
import jax.numpy as jnp
from jax import lax
from jax.experimental import pallas as pl
from jax.experimental.pallas import tpu as pltpu
from jax.experimental.pallas import tpu_sc as plsc

_NC = 2
_NS = 16
_B = 16384
_D = 64
_V = 1000000
_DPC = _D // _NC
_BPT = _B // _NS
_GC = 1024
_NGC = _BPT // _GC
_OW = _BPT + 16
_MAIN = 999936
_PC = _MAIN // 12
_TOFF = 1000064
_SLOT = _TOFF + 128
_REG = 0.0001


def _gather_chunks(src_sh, idx_v, dst_v, sem):
    cps = []
    for c in range(_NGC):
        sl = pl.ds(c * _GC, _GC)
        cps.append(pltpu.async_copy(src_sh.at[idx_v.at[sl]], dst_v.at[sl], sem))
    return cps


def _sc_body(idx_hbm, ut_hbm, it_hbm, utail_hbm, itail_hbm, out_hbm,
             idx_u, idx_p, idx_n, ue_v, pe_v, ne_v,
             acc_v, sq_v, row_sh, gsem, ssem):
    c = lax.axis_index("c")
    s = lax.axis_index("s")
    d0 = c * _DPC

    pltpu.sync_copy(idx_hbm.at[pl.ds(s * _BPT, _BPT)], idx_u)
    pltpu.sync_copy(idx_hbm.at[pl.ds(_B + s * _BPT, _BPT)], idx_p)
    pltpu.sync_copy(idx_hbm.at[pl.ds(2 * _B + s * _BPT, _BPT)], idx_n)

    def init_body(k, _):
        sl = pl.ds(k * 16, 16)
        for ref in (idx_u, idx_p, idx_n):
            iv = ref[sl]
            ref[sl] = jnp.where(iv < _MAIN, iv, iv + (_TOFF - _MAIN))
        acc_v[pl.ds(k * 16, 16)] = jnp.zeros((16,), jnp.float32)
        return 0
    lax.fori_loop(0, _BPT // 16, init_body, 0)

    def stage(tab, tail, d):
        @pl.when(s < 12)
        def _():
            sl = pl.ds(s * _PC, _PC)
            pltpu.async_copy(tab.at[d].at[sl], row_sh.at[sl], ssem)
        @pl.when(s == 12)
        def _():
            pltpu.async_copy(tail.at[d], row_sh.at[pl.ds(_TOFF, 128)], ssem)

    def wait_stage():
        @pl.when(s < 12)
        def _():
            sl = pl.ds(0, _PC)
            pltpu.make_async_copy(ut_hbm.at[0].at[sl], row_sh.at[sl], ssem).wait()
        @pl.when(s == 12)
        def _():
            pltpu.make_async_copy(
                utail_hbm.at[0], row_sh.at[pl.ds(_TOFF, 128)], ssem).wait()

    stage(ut_hbm, utail_hbm, d0)

    def dim_body(dl, acc_sq):
        d = d0 + dl

        wait_stage()
        plsc.subcore_barrier()
        for cp in _gather_chunks(row_sh, idx_u, ue_v, gsem):
            cp.wait()
        plsc.subcore_barrier()
        stage(it_hbm, itail_hbm, d)

        def squ_body(k, asq):
            ue = ue_v[pl.ds(k * 16, 16)]
            return asq + ue * ue
        acc_sq = lax.fori_loop(0, _BPT // 16, squ_body, acc_sq)

        wait_stage()
        plsc.subcore_barrier()
        cps = _gather_chunks(row_sh, idx_p, pe_v, gsem)
        cps += _gather_chunks(row_sh, idx_n, ne_v, gsem)
        for cp in cps:
            cp.wait()
        plsc.subcore_barrier()
        @pl.when(dl + 1 < _DPC)
        def _():
            stage(ut_hbm, utail_hbm, d + 1)

        def acc_body(k, asq):
            sl = pl.ds(k * 16, 16)
            ue = ue_v[sl]
            pe = pe_v[sl]
            ne = ne_v[sl]
            acc_v[sl] += ue * (pe - ne)
            return asq + pe * pe + ne * ne
        acc_sq = lax.fori_loop(0, _BPT // 16, acc_body, acc_sq)

        return acc_sq

    acc_sq = lax.fori_loop(0, _DPC, dim_body, jnp.zeros((16,), jnp.float32))

    sq_v[...] = acc_sq
    pltpu.sync_copy(acc_v, out_hbm.at[c, s, pl.ds(0, _BPT)])
    pltpu.sync_copy(sq_v, out_hbm.at[c, s, pl.ds(_BPT, 16)])


_sc_call = pl.kernel(
    _sc_body,
    out_type=jax.ShapeDtypeStruct((_NC, _NS, _OW), jnp.float32),
    mesh=plsc.VectorSubcoreMesh(core_axis_name="c", subcore_axis_name="s"),
    scratch_types=[
        pltpu.VMEM((_BPT,), jnp.int32),
        pltpu.VMEM((_BPT,), jnp.int32),
        pltpu.VMEM((_BPT,), jnp.int32),
        pltpu.VMEM((_BPT,), jnp.float32),
        pltpu.VMEM((_BPT,), jnp.float32),
        pltpu.VMEM((_BPT,), jnp.float32),
        pltpu.VMEM((_BPT,), jnp.float32),
        pltpu.VMEM((16,), jnp.float32),
        pltpu.VMEM_SHARED((_SLOT,), jnp.float32),
        pltpu.SemaphoreType.DMA,
        pltpu.SemaphoreType.DMA,
    ],
    compiler_params=pltpu.CompilerParams(needs_layout_passes=False),
)


def _tc_body(diff_ref, sq_ref, out_ref):
    x = diff_ref[0] + diff_ref[1]
    sg = 1.0 / (1.0 + jnp.exp(-x))
    l = jnp.log(sg + 1e-10)
    bpr = -jnp.sum(l) / _B
    reg = _REG * jnp.sum(sq_ref[...]) / _B
    out_ref[0, 0] = bpr + reg


def _tail128(table_t):
    t = lax.slice(table_t, (0, _MAIN), (_D, _V))
    return jnp.pad(t, ((0, 0), (0, 128 - (_V - _MAIN))))


def kernel(u, pi, ni, user_emb, item_emb):
    idx = jnp.concatenate([u, pi, ni])
    ut = user_emb.T
    it = item_emb.T
    part = _sc_call(idx, ut, it, _tail128(ut), _tail128(it))
    diff = part[:, :, :_BPT].reshape(2, 128, 128)
    sq = part[:, :, _BPT:].reshape(4, 128)
    out = pl.pallas_call(
        _tc_body,
        out_shape=jax.ShapeDtypeStruct((1, 1), jnp.float32),
        in_specs=[
            pl.BlockSpec(memory_space=pltpu.VMEM),
            pl.BlockSpec(memory_space=pltpu.VMEM),
        ],
        out_specs=pl.BlockSpec(memory_space=pltpu.SMEM),
    )(diff, sq)
    return out.reshape(())


# speedup vs baseline: 2.8141x; 1.0060x over previous
import jax
import jax.numpy as jnp
from jax import lax
from jax.experimental import pallas as pl
from jax.experimental.pallas import tpu as pltpu
from jax.experimental.pallas import tpu_sc as plsc

_NC = 2
_NS = 16
_B = 16384
_D = 64
_V = 1000000
_DPC = _D // _NC
_BPT = _B // _NS
_GC = 1024
_NGC = _BPT // _GC
_OW = _BPT + 16
_REG = 0.0001


def _gather_chunks(src_sh, idx_v, dst_v, sem):
    cps = []
    for c in range(_NGC):
        sl = pl.ds(c * _GC, _GC)
        cps.append(pltpu.async_copy(src_sh.at[idx_v.at[sl]], dst_v.at[sl], sem))
    return cps


def _sc_body(idx_hbm, ut_hbm, it_hbm, out_hbm,
             idx_u, idx_p, idx_n, ue_v, pe_v, ne_v,
             acc_v, sq_v, row_sh, gsem, ssem):
    c = lax.axis_index("c")
    s = lax.axis_index("s")
    d0 = c * _DPC

    pltpu.sync_copy(idx_hbm.at[pl.ds(s * _BPT, _BPT)], idx_u)
    pltpu.sync_copy(idx_hbm.at[pl.ds(_B + s * _BPT, _BPT)], idx_p)
    pltpu.sync_copy(idx_hbm.at[pl.ds(2 * _B + s * _BPT, _BPT)], idx_n)

    def zero_body(k, _):
        acc_v[pl.ds(k * 16, 16)] = jnp.zeros((16,), jnp.float32)
        return 0
    lax.fori_loop(0, _BPT // 16, zero_body, 0)

    @pl.when(s == 0)
    def _():
        pltpu.async_copy(ut_hbm.at[d0], row_sh, ssem)

    def wait_stage():
        pltpu.make_async_copy(ut_hbm.at[0], row_sh, ssem).wait()

    def dim_body(dl, acc_sq):
        d = d0 + dl

        @pl.when(s == 0)
        def _():
            wait_stage()
        plsc.subcore_barrier()
        for cp in _gather_chunks(row_sh, idx_u, ue_v, gsem):
            cp.wait()
        plsc.subcore_barrier()
        @pl.when(s == 0)
        def _():
            pltpu.async_copy(it_hbm.at[d], row_sh, ssem)

        def squ_body(k, asq):
            ue = ue_v[pl.ds(k * 16, 16)]
            return asq + ue * ue
        acc_sq = lax.fori_loop(0, _BPT // 16, squ_body, acc_sq)

        @pl.when(s == 0)
        def _():
            wait_stage()
        plsc.subcore_barrier()
        cps = _gather_chunks(row_sh, idx_p, pe_v, gsem)
        cps += _gather_chunks(row_sh, idx_n, ne_v, gsem)
        for cp in cps:
            cp.wait()
        plsc.subcore_barrier()
        @pl.when((s == 0) & (dl + 1 < _DPC))
        def _():
            pltpu.async_copy(ut_hbm.at[d + 1], row_sh, ssem)

        def acc_body(k, asq):
            sl = pl.ds(k * 16, 16)
            ue = ue_v[sl]
            pe = pe_v[sl]
            ne = ne_v[sl]
            acc_v[sl] += ue * (pe - ne)
            return asq + pe * pe + ne * ne
        acc_sq = lax.fori_loop(0, _BPT // 16, acc_body, acc_sq)

        return acc_sq

    acc_sq = lax.fori_loop(0, _DPC, dim_body, jnp.zeros((16,), jnp.float32))

    sq_v[...] = acc_sq
    pltpu.sync_copy(acc_v, out_hbm.at[c, s, pl.ds(0, _BPT)])
    pltpu.sync_copy(sq_v, out_hbm.at[c, s, pl.ds(_BPT, 16)])


_sc_call = pl.kernel(
    _sc_body,
    out_type=jax.ShapeDtypeStruct((_NC, _NS, _OW), jnp.float32),
    mesh=plsc.VectorSubcoreMesh(core_axis_name="c", subcore_axis_name="s"),
    scratch_types=[
        pltpu.VMEM((_BPT,), jnp.int32),
        pltpu.VMEM((_BPT,), jnp.int32),
        pltpu.VMEM((_BPT,), jnp.int32),
        pltpu.VMEM((_BPT,), jnp.float32),
        pltpu.VMEM((_BPT,), jnp.float32),
        pltpu.VMEM((_BPT,), jnp.float32),
        pltpu.VMEM((_BPT,), jnp.float32),
        pltpu.VMEM((16,), jnp.float32),
        pltpu.VMEM_SHARED((_V,), jnp.float32),
        pltpu.SemaphoreType.DMA,
        pltpu.SemaphoreType.DMA,
    ],
    compiler_params=pltpu.CompilerParams(needs_layout_passes=False),
)


def _tc_body(diff_ref, sq_ref, out_ref):
    x = diff_ref[0] + diff_ref[1]
    sg = 1.0 / (1.0 + jnp.exp(-x))
    l = jnp.log(sg + 1e-10)
    bpr = -jnp.sum(l) / _B
    reg = _REG * jnp.sum(sq_ref[...]) / _B
    out_ref[0, 0] = bpr + reg


def kernel(u, pi, ni, user_emb, item_emb):
    idx = jnp.concatenate([u, pi, ni])
    part = _sc_call(idx, user_emb.T, item_emb.T)
    diff = part[:, :, :_BPT].reshape(2, 128, 128)
    sq = part[:, :, _BPT:].reshape(4, 128)
    out = pl.pallas_call(
        _tc_body,
        out_shape=jax.ShapeDtypeStruct((1, 1), jnp.float32),
        in_specs=[
            pl.BlockSpec(memory_space=pltpu.VMEM),
            pl.BlockSpec(memory_space=pltpu.VMEM),
        ],
        out_specs=pl.BlockSpec(memory_space=pltpu.SMEM),
    )(diff, sq)
    return out.reshape(())
